# baseline (speedup 1.0000x reference)
import functools

import jax
import jax.numpy as jnp
from jax import lax
from jax.experimental import pallas as pl
from jax.experimental.pallas import tpu as pltpu

_E = 8
_R = 8
_NEG = -3.0e38


def _fused_kernel(x_ref, w65_ref, gw_ref, gb_ref, a2_ref, b2_ref,
                  gamma_ref, beta_ref, o_ref,
                  xc_ref, h_ref, xb_ref, s2_ref, delta_ref, wt_ref, st_ref,
                  sem, *, s_blk, ns, n_b, inv_s):
    b = pl.program_id(0)
    s = pl.program_id(1)
    D = x_ref.shape[2]
    slot = lax.rem(b, 2)
    pslot = 1 - slot

    def chunk_copy(bb, sl, j):
        return pltpu.make_async_copy(
            x_ref.at[bb, pl.ds(j * s_blk, s_blk), :],
            xc_ref.at[sl, pl.ds(j * s_blk, s_blk), :],
            sem.at[sl, j])

    @pl.when(jnp.logical_and(b == 0, s == 0))
    def _():
        for j in range(ns):
            chunk_copy(0, 0, j).start()

    @pl.when(jnp.logical_and(s == 0, jnp.logical_and(b >= 1, b < n_b)))
    def _():
        chunk_copy(b, slot, ns - 1).start()

    @pl.when(jnp.logical_and(s >= 1, b + 1 < n_b))
    def _():
        chunk_copy(b + 1, pslot, s - 1).start()

    @pl.when(b >= 1)
    def _normalize():
        @pl.when(s == 0)
        def _():
            h = h_ref[pslot] * inv_s
            logits = (jnp.sum(gw_ref[...] * h, axis=1, keepdims=True)
                      + gb_ref[...])
            iota8 = lax.broadcasted_iota(jnp.int32, (_E, 1), 0)
            m1 = jnp.max(logits)
            i1 = jnp.min(jnp.where(logits == m1, iota8, _E))
            masked = jnp.where(iota8 == i1, _NEG, logits)
            m2 = jnp.max(masked)
            i2 = jnp.min(jnp.where(masked == m2, iota8, _E))
            eb = jnp.exp(m2 - m1)
            denom = 1.0 + eb
            w1 = 1.0 / denom
            w2 = eb / denom
            t = jnp.sum(a2_ref[...] * h, axis=1, keepdims=True)
            e_ids = lax.broadcasted_iota(jnp.int32, (_E * _R, 1), 0) // _R
            wfull = (jnp.where(e_ids == i1, w1, 0.0)
                     + jnp.where(e_ids == i2, w2, 0.0))
            wt = wfull * t * (1.0 / _R)
            delta = jnp.sum(wt * b2_ref[...], axis=0, keepdims=True)
            delta_ref[...] = delta
            wt_ref[...] = wt.reshape(1, _E * _R)
            st_ref[0, 0] = jnp.sum(delta)
            st_ref[0, 1] = jnp.sum(delta * delta)

        xm = xc_ref[pslot, pl.ds(s * s_blk, s_blk), :]
        xb = xb_ref[pslot, pl.ds(s * s_blk, s_blk), :]
        s1 = xb[:, 64:65]
        cross = jnp.sum(xb[:, 0:64] * wt_ref[...], axis=1, keepdims=True)
        mu = (s1 + st_ref[0, 0]) * (1.0 / D)
        var = ((s2_ref[pslot, pl.ds(s * s_blk, s_blk), :] + 2.0 * cross
                + st_ref[0, 1]) * (1.0 / D) - mu * mu)
        rs = lax.rsqrt(var + 1e-5)
        o_ref[...] = (((xm + delta_ref[...] - mu) * rs) * gamma_ref[...]
                      + beta_ref[...])[None]
    @pl.when(b < n_b)
    def _stats():
        chunk_copy(b, slot, s).wait()
        xm = xc_ref[slot, pl.ds(s * s_blk, s_blk), :]

        @pl.when(s == 0)
        def _():
            h_ref[slot] = jnp.zeros((1, D), jnp.float32)

        h_ref[slot] += jnp.sum(xm, axis=0)[None, :]

        xb_ref[slot, pl.ds(s * s_blk, s_blk), :] = lax.dot_general(
            xm, w65_ref[...], (((1,), (0,)), ((), ())),
            preferred_element_type=jnp.float32)
        s2_ref[slot, pl.ds(s * s_blk, s_blk), :] = jnp.sum(
            xm * xm, axis=1, keepdims=True)


def kernel(x, gate_W, gate_b, A_all, B_all, gamma, beta):
    B, S, D = x.shape
    s_blk = 512
    ns = S // s_blk

    A2 = A_all.reshape(_E * _R, D)
    B2 = jnp.transpose(B_all, (0, 2, 1)).reshape(_E * _R, D)
    w65 = jnp.concatenate([B2.T, jnp.ones((D, 1), jnp.float32)], axis=1)
    gb = gate_b.reshape(_E, 1)
    gm = gamma.reshape(1, D)
    bt = beta.reshape(1, D)

    out = pl.pallas_call(
        functools.partial(_fused_kernel, s_blk=s_blk, ns=ns, n_b=B,
                          inv_s=1.0 / S),
        grid=(B + 1, ns),
        in_specs=[
            pl.BlockSpec(memory_space=pl.ANY),
            pl.BlockSpec((D, _E * _R + 1), lambda b, s: (0, 0)),
            pl.BlockSpec((_E, D), lambda b, s: (0, 0)),
            pl.BlockSpec((_E, 1), lambda b, s: (0, 0)),
            pl.BlockSpec((_E * _R, D), lambda b, s: (0, 0)),
            pl.BlockSpec((_E * _R, D), lambda b, s: (0, 0)),
            pl.BlockSpec((1, D), lambda b, s: (0, 0)),
            pl.BlockSpec((1, D), lambda b, s: (0, 0)),
        ],
        out_specs=pl.BlockSpec(
            (1, s_blk, D),
            lambda b, s: (jnp.maximum(b - 1, 0), s * jnp.minimum(b, 1), 0)),
        out_shape=jax.ShapeDtypeStruct((B, S, D), jnp.float32),
        scratch_shapes=[
            pltpu.VMEM((2, S, D), jnp.float32),
            pltpu.VMEM((2, 1, D), jnp.float32),
            pltpu.VMEM((2, S, _E * _R + 1), jnp.float32),
            pltpu.VMEM((2, S, 1), jnp.float32),
            pltpu.VMEM((1, D), jnp.float32),
            pltpu.VMEM((1, _E * _R), jnp.float32),
            pltpu.SMEM((1, 2), jnp.float32),
            pltpu.SemaphoreType.DMA((2, S // s_blk)),
        ],
    )(x, w65, gate_W, gb, A2, B2, gm, bt)
    return out


# speedup vs baseline: 1.0188x; 1.0188x over previous
import functools

import jax
import jax.numpy as jnp
from jax import lax
from jax.experimental import pallas as pl
from jax.experimental.pallas import tpu as pltpu

_E = 8
_R = 8
_NEG = -3.0e38


def _fused_kernel(x_ref, w65_ref, gw_ref, gb_ref, a2_ref, b2_ref,
                  gamma_ref, beta_ref, o_ref,
                  xc_ref, h_ref, xb_ref, s2_ref, delta_ref, wt_ref, st_ref,
                  sem, *, s_blk, ns, n_b, inv_s):
    b = pl.program_id(0)
    s = pl.program_id(1)
    D = x_ref.shape[2]
    slot = lax.rem(b, 2)
    pslot = 1 - slot

    def chunk_copy(bb, sl, j):
        return pltpu.make_async_copy(
            x_ref.at[bb, pl.ds(j * s_blk, s_blk), :],
            xc_ref.at[sl, pl.ds(j * s_blk, s_blk), :],
            sem.at[sl, j])

    @pl.when(jnp.logical_and(b == 0, s == 0))
    def _():
        for j in range(ns):
            chunk_copy(0, 0, j).start()

    @pl.when(jnp.logical_and(s == 0, jnp.logical_and(b >= 1, b < n_b)))
    def _():
        chunk_copy(b, slot, ns - 1).start()

    @pl.when(jnp.logical_and(s >= 1, b + 1 < n_b))
    def _():
        chunk_copy(b + 1, pslot, s - 1).start()

    @pl.when(b >= 1)
    def _normalize():
        @pl.when(s == 0)
        def _():
            hp = h_ref[pslot] * inv_s
            logits = hp[:, 0:8] + gb_ref[...]
            iota8 = lax.broadcasted_iota(jnp.int32, (1, _E), 1)
            m1 = jnp.max(logits)
            i1 = jnp.min(jnp.where(logits == m1, iota8, _E))
            masked = jnp.where(iota8 == i1, _NEG, logits)
            m2 = jnp.max(masked)
            i2 = jnp.min(jnp.where(masked == m2, iota8, _E))
            eb = jnp.exp(m2 - m1)
            denom = 1.0 + eb
            w1 = 1.0 / denom
            w2 = eb / denom
            t = hp[:, 8:72]
            e_ids = lax.broadcasted_iota(jnp.int32, (1, _E * _R), 1) // _R
            wfull = (jnp.where(e_ids == i1, w1, 0.0)
                     + jnp.where(e_ids == i2, w2, 0.0))
            wt = wfull * t * (1.0 / _R)
            wt_ref[...] = wt
            delta = lax.dot_general(wt, b2_ref[...], (((1,), (0,)), ((), ())),
                                    preferred_element_type=jnp.float32)
            delta_ref[...] = delta
            st_ref[0, 0] = jnp.sum(delta)
            st_ref[0, 1] = jnp.sum(delta * delta)

        xm = xc_ref[pslot, pl.ds(s * s_blk, s_blk), :]
        xb = xb_ref[pslot, pl.ds(s * s_blk, s_blk), :]
        s1 = xb[:, 64:65]
        cross = jnp.sum(xb[:, 0:64] * wt_ref[...], axis=1, keepdims=True)
        mu = (s1 + st_ref[0, 0]) * (1.0 / D)
        var = ((s2_ref[pslot, pl.ds(s * s_blk, s_blk), :] + 2.0 * cross
                + st_ref[0, 1]) * (1.0 / D) - mu * mu)
        rs = lax.rsqrt(var + 1e-5)
        o_ref[...] = (((xm + delta_ref[...] - mu) * rs) * gamma_ref[...]
                      + beta_ref[...])[None]
    @pl.when(b < n_b)
    def _stats():
        chunk_copy(b, slot, s).wait()
        xm = xc_ref[slot, pl.ds(s * s_blk, s_blk), :]

        @pl.when(s == 0)
        def _():
            h_ref[slot] = jnp.zeros((1, 72), jnp.float32)

        xw = lax.dot_general(xm, w65_ref[...], (((1,), (0,)), ((), ())),
                             preferred_element_type=jnp.float32)
        xb_ref[slot, pl.ds(s * s_blk, s_blk), :] = xw[:, 0:65]
        h_ref[slot] += jnp.sum(xw[:, 65:137], axis=0)[None, :]
        s2_ref[slot, pl.ds(s * s_blk, s_blk), :] = jnp.sum(
            xm * xm, axis=1, keepdims=True)


def kernel(x, gate_W, gate_b, A_all, B_all, gamma, beta):
    B, S, D = x.shape
    s_blk = 512
    ns = S // s_blk

    A2 = A_all.reshape(_E * _R, D)
    B2 = jnp.transpose(B_all, (0, 2, 1)).reshape(_E * _R, D)
    w65 = jnp.concatenate(
        [B2.T, jnp.ones((D, 1), jnp.float32), gate_W.T, A2.T], axis=1)
    gb = gate_b.reshape(1, _E)
    gm = gamma.reshape(1, D)
    bt = beta.reshape(1, D)

    out = pl.pallas_call(
        functools.partial(_fused_kernel, s_blk=s_blk, ns=ns, n_b=B,
                          inv_s=1.0 / S),
        grid=(B + 1, ns),
        in_specs=[
            pl.BlockSpec(memory_space=pl.ANY),
            pl.BlockSpec((D, 2 * _E * _R + _E + 1), lambda b, s: (0, 0)),
            pl.BlockSpec((_E, D), lambda b, s: (0, 0)),
            pl.BlockSpec((1, _E), lambda b, s: (0, 0)),
            pl.BlockSpec((_E * _R, D), lambda b, s: (0, 0)),
            pl.BlockSpec((_E * _R, D), lambda b, s: (0, 0)),
            pl.BlockSpec((1, D), lambda b, s: (0, 0)),
            pl.BlockSpec((1, D), lambda b, s: (0, 0)),
        ],
        out_specs=pl.BlockSpec(
            (1, s_blk, D),
            lambda b, s: (jnp.maximum(b - 1, 0), s * jnp.minimum(b, 1), 0)),
        out_shape=jax.ShapeDtypeStruct((B, S, D), jnp.float32),
        scratch_shapes=[
            pltpu.VMEM((2, S, D), jnp.float32),
            pltpu.VMEM((2, 1, 72), jnp.float32),
            pltpu.VMEM((2, S, _E * _R + 1), jnp.float32),
            pltpu.VMEM((2, S, 1), jnp.float32),
            pltpu.VMEM((1, D), jnp.float32),
            pltpu.VMEM((1, _E * _R), jnp.float32),
            pltpu.SMEM((1, 2), jnp.float32),
            pltpu.SemaphoreType.DMA((2, S // s_blk)),
        ],
    )(x, w65, gate_W, gb, A2, B2, gm, bt)
    return out
